# baseline (speedup 1.0000x reference)
import functools

import jax
import jax.numpy as jnp
from jax import lax
from jax.experimental import pallas as pl
from jax.experimental.pallas import tpu as pltpu
from jax.experimental.pallas import tpu_sc as plsc

_B = 4096
_D = 64
_NC, _NS = 2, 16
_NW = _NC * _NS
_TPW = _B // _NW
_CH = 32
_NCHUNK = _TPW // _CH
_RPT = 8
_MARGIN = 1.0
_L = 16


def _rsqrt(x):
    i = lax.bitcast_convert_type(x, jnp.int32)
    i = jnp.int32(0x5F3759DF) - lax.shift_right_logical(i, 1)
    y = lax.bitcast_convert_type(i, jnp.float32)
    xh = x * 0.5
    for _ in range(3):
        y = y * (1.5 - xh * y * y)
    return y


def _sumall(v):
    return jnp.full((_L,), jnp.sum(v), jnp.float32)


def _scale(s2):
    norm = s2 * _rsqrt(s2)
    return jnp.minimum(1.0, 1.0 / (norm + 1e-7))


def _sc_body(entidx_hbm, relidx_hbm, eE_hbm, eP_hbm, rE_hbm, rP_hbm,
             out_hbm, ei00, ei01, ei10, ei11, ri0, ri1,
             eE_v, eP_v, rE_v, rP_v, out_v, sem0, sem1):
    wid = lax.axis_index("s") * _NC + lax.axis_index("c")
    sems = (sem0, sem1)
    eidx = ((ei00, ei01), (ei10, ei11))
    ridx = (ri0, ri1)
    handles = {}

    def issue(chunk, buf):
        base_t = wid * _TPW + chunk * _CH
        hs = []
        for j in range(2):
            pltpu.sync_copy(
                entidx_hbm.at[pl.ds(base_t * _RPT + j * 128, 128)],
                eidx[buf][j])
            dst_e = eE_v.at[buf, pl.ds(j * 128, 128), :]
            dst_p = eP_v.at[buf, pl.ds(j * 128, 128), :]
            hs.append(pltpu.async_copy(eE_hbm.at[eidx[buf][j]], dst_e,
                                       sems[buf]))
            hs.append(pltpu.async_copy(eP_hbm.at[eidx[buf][j]], dst_p,
                                       sems[buf]))
        pltpu.sync_copy(relidx_hbm.at[pl.ds(base_t, _CH)], ridx[buf])
        hs.append(pltpu.async_copy(rE_hbm.at[ridx[buf]], rE_v.at[buf],
                                   sems[buf]))
        hs.append(pltpu.async_copy(rP_hbm.at[ridx[buf]], rP_v.at[buf],
                                   sems[buf]))
        handles[buf] = hs

    def row(ref, r0):
        return [ref[r0, pl.ds(c * _L, _L)] for c in range(_D // _L)]

    acc = jnp.zeros((_L,), jnp.float32)
    issue(0, 0)
    for chunk in range(_NCHUNK):
        buf = chunk % 2
        if chunk + 1 < _NCHUNK:
            issue(chunk + 1, 1 - buf)
        for h in handles[buf]:
            h.wait()
        eEb, ePb = eE_v.at[buf], eP_v.at[buf]
        rEb, rPb = rE_v.at[buf], rP_v.at[buf]

        def trip(i, acc):
            aE, s_ent = [], []
            for k in range(_RPT):
                r0 = i * _RPT + k
                e = row(eEb, r0)
                p = row(ePb, r0)
                s2e = e[0] * e[0]
                s2p = p[0] * p[0]
                d = e[0] * p[0]
                for c in range(1, _D // _L):
                    s2e += e[c] * e[c]
                    s2p += p[c] * p[c]
                    d += e[c] * p[c]
                ae = _scale(_sumall(s2e))
                ap = _scale(_sumall(s2p))
                aE.append(ae)
                s_ent.append(ae * ap * _sumall(d))
            re = row(rEb, i)
            rp = row(rPb, i)
            s2re = re[0] * re[0]
            s2rp = rp[0] * rp[0]
            for c in range(1, _D // _L):
                s2re += re[c] * re[c]
                s2rp += rp[c] * rp[c]
            ar = _scale(_sumall(s2re))
            arp = _scale(_sumall(s2rp))
            rv = [re[c] * ar + 1e-6 for c in range(_D // _L)]
            rps = [rp[c] * arp for c in range(_D // _L)]
            dists = []
            for (hk, tk) in ((0, 1), (2, 5), (3, 6), (4, 7)):
                dS = s_ent[hk] - s_ent[tk]
                eh = row(eEb, i * _RPT + hk)
                et = row(eEb, i * _RPT + tk)
                ds2 = None
                for c in range(_D // _L):
                    dv = (eh[c] * aE[hk] - et[c] * aE[tk]
                          + rps[c] * dS + rv[c])
                    ds2 = dv * dv if ds2 is None else ds2 + dv * dv
                s2 = _sumall(ds2)
                dists.append(s2 * _rsqrt(s2))
            neg_mean = (dists[1] + dists[2] + dists[3]) * (1.0 / 3.0)
            loss = jnp.maximum(dists[0] - neg_mean + _MARGIN, 0.0)
            return acc + loss

        acc = lax.fori_loop(0, _CH, trip, acc)

    out_v[...] = acc
    pltpu.sync_copy(out_v, out_hbm.at[wid])


@jax.jit
def _sc_call(entidx, relidx, eE, eP, rE, rP):
    mesh = plsc.VectorSubcoreMesh(core_axis_name="c", subcore_axis_name="s",
                                  num_cores=_NC, num_subcores=_NS)
    f = pl.kernel(
        _sc_body,
        out_type=jax.ShapeDtypeStruct((_NW, _L), jnp.float32),
        mesh=mesh,
        compiler_params=pltpu.CompilerParams(needs_layout_passes=False,
                                             use_tc_tiling_on_sc=False),
        scratch_types=[
            pltpu.VMEM((128,), jnp.int32),
            pltpu.VMEM((128,), jnp.int32),
            pltpu.VMEM((128,), jnp.int32),
            pltpu.VMEM((128,), jnp.int32),
            pltpu.VMEM((_CH,), jnp.int32),
            pltpu.VMEM((_CH,), jnp.int32),
            pltpu.VMEM((2, _CH * _RPT, _D), jnp.float32),
            pltpu.VMEM((2, _CH * _RPT, _D), jnp.float32),
            pltpu.VMEM((2, _CH, _D), jnp.float32),
            pltpu.VMEM((2, _CH, _D), jnp.float32),
            pltpu.VMEM((_L,), jnp.float32),
            pltpu.SemaphoreType.DMA,
            pltpu.SemaphoreType.DMA,
        ],
    )
    return f(entidx, relidx, eE, eP, rE, rP)


def kernel(triplets, neg, entityEmb, entityEmbP, relationEmb, relationEmbP):
    h = triplets[:, 0:1].astype(jnp.int32)
    t = triplets[:, 2:3].astype(jnp.int32)
    r = triplets[:, 1].astype(jnp.int32)
    nh = neg[:, :, 0].astype(jnp.int32)
    nt = neg[:, :, 2].astype(jnp.int32)
    ent = jnp.concatenate([h, t, nh, nt], axis=1).reshape(-1)
    out = _sc_call(ent, r, entityEmb, entityEmbP, relationEmb, relationEmbP)
    return jnp.sum(out[:, 0]) / _B
